# baseline (speedup 1.0000x reference)
import functools

import jax
import jax.numpy as jnp
from jax import lax
from jax.experimental import pallas as pl
from jax.experimental.pallas import tpu as pltpu
from jax.experimental.pallas import tpu_sc as plsc

_B = 4096
_L = 200
_D = 128
_NW = 32
_HL = 100
_NSEQ = _B * _L // _HL // _NW
_NCH = _D // 16
_NS = 8


def _sc_body(x_hbm, tab_hbm, out_hbm, tab_v, buf, insem, outsem):
    wid = lax.axis_index("s") * 2 + lax.axis_index("c")
    base = wid * _NSEQ
    pltpu.sync_copy(tab_hbm, tab_v)
    t0 = [tab_v[0, pl.ds(16 * j, 16)] for j in range(_NCH)]

    def in_cp(s, q):
        return pltpu.make_async_copy(x_hbm.at[base + s], buf.at[pl.ds(q * _HL, _HL)], insem)

    def out_cp(s, q):
        return pltpu.make_async_copy(buf.at[pl.ds(q * _HL, _HL)], out_hbm.at[base + s], outsem)

    def compute(q, hoff):
        bufp = buf.at[pl.ds(q * _HL, _HL)]

        @plsc.parallel_loop(0, _HL, 1, unroll=2)
        def _row(l):
            xs = [bufp[l, pl.ds(16 * j, 16)] for j in range(_NCH)]
            nz = xs[0] != 0.0
            for v in xs[1:]:
                nz = nz | (v != 0.0)
            cnt = plsc.all_reduce_population_count(nz)
            m = cnt > 0
            for j in range(_NCH):
                t = jnp.where(m, tab_v[hoff + l + 1, pl.ds(16 * j, 16)], t0[j])
                bufp[l, pl.ds(16 * j, 16)] = xs[j] + t

    for q in range(_NS - 1):
        in_cp(q, q).start()

    def seq_body(k, carry):
        for q in range(_NS):
            s = _NS * k + q
            in_cp(s, q).wait()
            compute(q, (s % 2) * _HL)
            out_cp(s, q).start()
            if q == 0:
                @pl.when(k > 0)
                def _():
                    out_cp(s - 1, _NS - 1).wait()
            else:
                out_cp(s - 1, q - 1).wait()

            @pl.when(s + _NS - 1 < _NSEQ)
            def _():
                in_cp(s + _NS - 1, (q + _NS - 1) % _NS).start()
        return carry

    lax.fori_loop(0, _NSEQ // _NS, seq_body, 0)
    out_cp(_NSEQ - 1, (_NSEQ - 1) % _NS).wait()


def kernel(x, pos_table):
    B, L, D = x.shape
    x2 = x.reshape(B * L // _HL, _HL, D)
    mesh = plsc.VectorSubcoreMesh(core_axis_name="c", subcore_axis_name="s")
    run = functools.partial(
        pl.kernel,
        mesh=mesh,
        compiler_params=pltpu.CompilerParams(needs_layout_passes=False),
        out_type=jax.ShapeDtypeStruct((B * L // _HL, _HL, D), jnp.float32),
        scratch_types=[
            pltpu.VMEM((L + 1, D), jnp.float32),
            pltpu.VMEM((_NS * _HL, D), jnp.float32),
            pltpu.SemaphoreType.DMA,
            pltpu.SemaphoreType.DMA,
        ],
    )(_sc_body)
    return run(x2, pos_table).reshape(B, L, D)


# speedup vs baseline: 3.3953x; 3.3953x over previous
import functools

import jax
import jax.numpy as jnp
from jax import lax
from jax.experimental import pallas as pl
from jax.experimental.pallas import tpu as pltpu
from jax.experimental.pallas import tpu_sc as plsc

_B = 4096
_L = 200
_D = 128
_NW = 32
_SEQ_PER_W = _B // _NW


_NCH = _D // 16
_NS = 4


def _sc_body(x_hbm, tab_hbm, out_hbm, tab_v, buf, insem, outsem):
    wid = lax.axis_index("s") * 2 + lax.axis_index("c")
    base = wid * _SEQ_PER_W
    pltpu.sync_copy(tab_hbm, tab_v)
    t0 = [tab_v[0, pl.ds(16 * j, 16)] for j in range(_NCH)]

    def in_cp(s):
        return pltpu.make_async_copy(x_hbm.at[base + s], buf.at[s % _NS], insem)

    def out_cp(s):
        return pltpu.make_async_copy(buf.at[s % _NS], out_hbm.at[base + s], outsem)

    def in_cp_q(s, q):
        return pltpu.make_async_copy(x_hbm.at[base + s], buf.at[q], insem)

    def out_cp_q(s, q):
        return pltpu.make_async_copy(buf.at[q], out_hbm.at[base + s], outsem)

    def compute(q):
        bufp = buf.at[q]

        @plsc.parallel_loop(0, _L, 1, unroll=2)
        def _row(l):
            xs = [bufp[l, pl.ds(16 * j, 16)] for j in range(_NCH)]
            nz = xs[0] != 0.0
            for v in xs[1:]:
                nz = nz | (v != 0.0)
            cnt = plsc.all_reduce_population_count(nz)
            m = cnt > 0
            for j in range(_NCH):
                t = jnp.where(m, tab_v[l + 1, pl.ds(16 * j, 16)], t0[j])
                bufp[l, pl.ds(16 * j, 16)] = xs[j] + t

    in_cp(0).start()
    in_cp(1).start()
    in_cp(2).start()

    def seq_body4(k, carry):
        for q in range(_NS):
            s = _NS * k + q
            in_cp_q(s, q).wait()
            compute(q)
            out_cp_q(s, q).start()
            if q == 0:
                @pl.when(k > 0)
                def _():
                    out_cp_q(s - 1, _NS - 1).wait()
            else:
                out_cp_q(s - 1, q - 1).wait()

            @pl.when(s + _NS - 1 < _SEQ_PER_W)
            def _():
                in_cp_q(s + _NS - 1, (q + _NS - 1) % _NS).start()
        return carry

    lax.fori_loop(0, _SEQ_PER_W // _NS, seq_body4, 0)
    out_cp(_SEQ_PER_W - 1).wait()


def kernel(x, pos_table):
    B, L, D = x.shape
    mesh = plsc.VectorSubcoreMesh(core_axis_name="c", subcore_axis_name="s")
    run = functools.partial(
        pl.kernel,
        mesh=mesh,
        compiler_params=pltpu.CompilerParams(needs_layout_passes=False),
        out_type=jax.ShapeDtypeStruct((B, L, D), jnp.float32),
        scratch_types=[
            pltpu.VMEM((L + 1, D), jnp.float32),
            pltpu.VMEM((_NS, L, D), jnp.float32),
            pltpu.SemaphoreType.DMA,
            pltpu.SemaphoreType.DMA,
        ],
    )(_sc_body)
    return run(x, pos_table)


# speedup vs baseline: 3.4061x; 1.0032x over previous
import functools

import jax
import jax.numpy as jnp
from jax import lax
from jax.experimental import pallas as pl
from jax.experimental.pallas import tpu as pltpu
from jax.experimental.pallas import tpu_sc as plsc

_B = 4096
_L = 200
_D = 128
_NW = 32
_SEQ_PER_W = _B // _NW
_NCH = _D // 16
_NS = 4


def _sc_body(x_hbm, tab_hbm, out_hbm, tab_v, buf, insem, outsem):
    wid = lax.axis_index("s") * 2 + lax.axis_index("c")
    base = wid * _SEQ_PER_W
    pltpu.sync_copy(tab_hbm, tab_v)
    t0 = [tab_v[0, pl.ds(16 * j, 16)] for j in range(_NCH)]

    def in_cp(s, q):
        return pltpu.make_async_copy(x_hbm.at[base + s], buf.at[q], insem)

    def out_cp(s, q):
        return pltpu.make_async_copy(buf.at[q], out_hbm.at[base + s], outsem)

    def compute(q):
        bufp = buf.at[q]

        @plsc.parallel_loop(0, _L, 1, unroll=2)
        def _row(l):
            xs = [bufp[l, pl.ds(16 * j, 16)] for j in range(_NCH)]
            nz = xs[0] != 0.0
            for v in xs[1:]:
                nz = nz | (v != 0.0)
            cnt = plsc.all_reduce_population_count(nz)
            m = cnt > 0
            for j in range(_NCH):
                t = jnp.where(m, tab_v[l + 1, pl.ds(16 * j, 16)], t0[j])
                bufp[l, pl.ds(16 * j, 16)] = xs[j] + t

    for q in range(_NS - 1):
        in_cp(q, q).start()

    def seq_body(k, carry):
        for q in range(_NS):
            s = _NS * k + q
            in_cp(s, q).wait()
            compute(q)
            out_cp(s, q).start()
            if q == 0:
                @pl.when(k > 0)
                def _():
                    out_cp(s - 1, _NS - 1).wait()
            else:
                out_cp(s - 1, q - 1).wait()

            @pl.when(s + _NS - 1 < _SEQ_PER_W)
            def _():
                in_cp(s + _NS - 1, (q + _NS - 1) % _NS).start()
        return carry

    lax.fori_loop(0, _SEQ_PER_W // _NS, seq_body, 0)
    out_cp(_SEQ_PER_W - 1, (_SEQ_PER_W - 1) % _NS).wait()


def kernel(x, pos_table):
    B, L, D = x.shape
    mesh = plsc.VectorSubcoreMesh(core_axis_name="c", subcore_axis_name="s")
    run = functools.partial(
        pl.kernel,
        mesh=mesh,
        compiler_params=pltpu.CompilerParams(needs_layout_passes=False),
        out_type=jax.ShapeDtypeStruct((B, L, D), jnp.float32),
        scratch_types=[
            pltpu.VMEM((L + 1, D), jnp.float32),
            pltpu.VMEM((_NS, L, D), jnp.float32),
            pltpu.SemaphoreType.DMA,
            pltpu.SemaphoreType.DMA,
        ],
    )(_sc_body)
    return run(x, pos_table)
